# Initial kernel scaffold; baseline (speedup 1.0000x reference)
import jax
import jax.numpy as jnp
from jax.experimental import pallas as pl


def kernel(node_features, edge_index, edge_features, W_np, b_np, g_np, beta_np, W_ep, b_ep, g_ep, beta_ep, Wg, att_s, att_d, W_e, att_e, bias_g, lnG, lnB):
    raise NotImplementedError("write your pallas kernel here")


# speedup vs baseline: 22.7179x; 22.7179x over previous
import functools

import jax
import jax.numpy as jnp
from jax import lax
from jax.experimental import pallas as pl
from jax.experimental.pallas import tpu as pltpu
from jax.experimental.pallas import tpu_sc as plsc

F32 = jnp.float32

N_NODES = 10000
N_EDGES = 320000
D = 128
H = 8
NC, NS = 2, 16
NW = NC * NS
EPW = N_EDGES // NW
K = 80
NCH = EPW // K
NP = 10240
NR = NP // NS


def _ln(h, g, b):
    m = jnp.mean(h, axis=-1, keepdims=True)
    v = jnp.mean((h - m) ** 2, axis=-1, keepdims=True)
    return (h - m) * lax.rsqrt(v + 1e-5) * g + b


def _node_prep_body(nf, W, b, g, be, Wg, Ps, Pd, xw_o, as_o, ad_o):
    x = jnp.maximum(_ln(jnp.dot(nf[...], W[...], preferred_element_type=F32)
                        + b[...], g[...], be[...]), 0.0)
    xw = jnp.dot(x, Wg[...], preferred_element_type=F32)
    xw_o[...] = xw
    as_o[...] = jnp.dot(xw, Ps[...], preferred_element_type=F32)
    ad_o[...] = jnp.dot(xw, Pd[...], preferred_element_type=F32)


def _node_prep(nf, W, b, g, be, Wg, Ps16, Pd16):
    R = 400
    grid = (N_NODES // R,)
    full = lambda s: pl.BlockSpec(s, lambda i: (0, 0))
    return pl.pallas_call(
        _node_prep_body,
        grid=grid,
        in_specs=[pl.BlockSpec((R, D), lambda i: (i, 0)),
                  full((D, D)), full((1, D)), full((1, D)), full((1, D)),
                  full((D, D)), full((D, D)), full((D, D))],
        out_specs=[pl.BlockSpec((R, D), lambda i: (i, 0)),
                   pl.BlockSpec((R, D), lambda i: (i, 0)),
                   pl.BlockSpec((R, D), lambda i: (i, 0))],
        out_shape=[jax.ShapeDtypeStruct((N_NODES, D), F32),
                   jax.ShapeDtypeStruct((N_NODES, D), F32),
                   jax.ShapeDtypeStruct((N_NODES, D), F32)],
    )(nf, W, b, g, be, Wg, Ps16, Pd16)


def _edge_prep_body(ef, W, b, g, be, V0, V1, a0_o, a1_o):
    ea = jnp.maximum(_ln(jnp.dot(ef[...], W[...], preferred_element_type=F32)
                         + b[...], g[...], be[...]), 0.0)
    a0_o[...] = jnp.dot(ea, V0[...], preferred_element_type=F32)
    a1_o[...] = jnp.dot(ea, V1[...], preferred_element_type=F32)


def _edge_prep(ef, W, b, g, be, V016, V116):
    R = 640
    grid = (N_EDGES // R,)
    DE = ef.shape[1]
    full = lambda s: pl.BlockSpec(s, lambda i: (0, 0))
    return pl.pallas_call(
        _edge_prep_body,
        grid=grid,
        in_specs=[pl.BlockSpec((R, DE), lambda i: (i, 0)),
                  full((DE, D)), full((1, D)), full((1, D)), full((1, D)),
                  full((D, 16)), full((D, 16))],
        out_specs=[pl.BlockSpec((R, 16), lambda i: (i, 0)),
                   pl.BlockSpec((R, 16), lambda i: (i, 0))],
        out_shape=[jax.ShapeDtypeStruct((N_EDGES, 16), F32),
                   jax.ShapeDtypeStruct((N_EDGES, 16), F32)],
    )(ef, W, b, g, be, V016, V116)


def _finish0_body(op, dp, Ex, bias, g, be, Wg, Ps, Pd, xw_o, as_o, ad_o):
    p = op[0] + op[1]
    den16 = (dp[0] + dp[1])[:, :16]
    den = jnp.dot(den16, Ex[...], preferred_element_type=F32) + 1e-16
    x1 = jnp.maximum(_ln(p / den + bias[...], g[...], be[...]), 0.0)
    xin = x1 + x1
    xw = jnp.dot(xin, Wg[...], preferred_element_type=F32)
    xw_o[...] = xw
    as_o[...] = jnp.dot(xw, Ps[...], preferred_element_type=F32)
    ad_o[...] = jnp.dot(xw, Pd[...], preferred_element_type=F32)


def _finish0(outp, denp, Ex16, bias, g, be, Wg, Ps16, Pd16):
    R = 400
    grid = (N_NODES // R,)
    full = lambda s: pl.BlockSpec(s, lambda i: (0, 0))
    return pl.pallas_call(
        _finish0_body,
        grid=grid,
        in_specs=[pl.BlockSpec((NC, R, D), lambda i: (0, i, 0)),
                  pl.BlockSpec((NC, R, D), lambda i: (0, i, 0)),
                  full((16, D)), full((1, D)), full((1, D)), full((1, D)),
                  full((D, D)), full((D, D)), full((D, D))],
        out_specs=[pl.BlockSpec((R, D), lambda i: (i, 0)),
                   pl.BlockSpec((R, D), lambda i: (i, 0)),
                   pl.BlockSpec((R, D), lambda i: (i, 0))],
        out_shape=[jax.ShapeDtypeStruct((N_NODES, D), F32),
                   jax.ShapeDtypeStruct((N_NODES, D), F32),
                   jax.ShapeDtypeStruct((N_NODES, D), F32)],
    )(outp, denp, Ex16, bias, g, be, Wg, Ps16, Pd16)


def _finish1_body(op, dp, Ex, bias, g, be, x_o, gemb_o):
    p = op[0] + op[1]
    den16 = (dp[0] + dp[1])[:, :16]
    den = jnp.dot(den16, Ex[...], preferred_element_type=F32) + 1e-16
    x = _ln(p / den + bias[...], g[...], be[...])
    x_o[...] = x

    @pl.when(pl.program_id(0) == 0)
    def _():
        gemb_o[...] = jnp.zeros_like(gemb_o)

    gemb_o[...] += jnp.sum(x, axis=0, keepdims=True) * (1.0 / N_NODES)


def _finish1(outp, denp, Ex16, bias, g, be):
    R = 400
    grid = (N_NODES // R,)
    full = lambda s: pl.BlockSpec(s, lambda i: (0, 0))
    return pl.pallas_call(
        _finish1_body,
        grid=grid,
        in_specs=[pl.BlockSpec((NC, R, D), lambda i: (0, i, 0)),
                  pl.BlockSpec((NC, R, D), lambda i: (0, i, 0)),
                  full((16, D)), full((1, D)), full((1, D)), full((1, D))],
        out_specs=[pl.BlockSpec((R, D), lambda i: (i, 0)),
                   pl.BlockSpec((1, D), lambda i: (0, 0))],
        out_shape=[jax.ShapeDtypeStruct((N_NODES, D), F32),
                   jax.ShapeDtypeStruct((1, D), F32)],
    )(outp, denp, Ex16, bias, g, be)


_GATHER_DNUMS = lax.GatherDimensionNumbers(
    offset_dims=(), collapsed_slice_dims=(0,), start_index_map=(0,))


def _lane_splat(v, j):
    idx = jnp.full((16, 1), j, jnp.int32)
    return lax.gather(v, idx, _GATHER_DNUMS, (1,),
                      mode=lax.GatherScatterMode.PROMISE_IN_BOUNDS)

KS = 40
NCHS = EPW // KS


def _scores_sc_body(src_h, dstN_h, dst_h, ae_h, T_h, ex_h, denp_h,
                    idx2_v, dst_v, ae_v, sg, ex_v, ex128, zbufd, dacc, s1):
    cid = lax.axis_index("c")
    sid = lax.axis_index("s")
    wid = cid * NS + sid

    def zrow(i, _):
        for j in range(D // 16):
            zbufd[i, pl.ds(j * 16, 16)] = jnp.zeros((16,), F32)
        return 0

    lax.fori_loop(0, 128, zrow, 0)

    def zrow2(i, _):
        for j in range(1, D // 16):
            ex128[i, pl.ds(j * 16, 16)] = jnp.zeros((16,), F32)
        return 0

    lax.fori_loop(0, KS, zrow2, 0)
    for k2 in range(NR // 128):
        pltpu.sync_copy(zbufd, dacc.at[pl.ds(sid * NR + k2 * 128, 128)])
    plsc.subcore_barrier()

    def chunk(t, _):
        base = wid * EPW + t * KS
        pltpu.sync_copy(src_h.at[pl.ds(base, KS)], idx2_v.at[pl.ds(0, KS)])
        pltpu.sync_copy(dstN_h.at[pl.ds(base, KS)], idx2_v.at[pl.ds(KS, KS)])
        pltpu.sync_copy(dst_h.at[pl.ds(base, KS)], dst_v)
        pltpu.sync_copy(ae_h.at[pl.ds(base, KS)], ae_v)
        c1 = pltpu.async_copy(T_h.at[idx2_v], sg, s1)
        c1.wait()

        def erow(e, _):
            a = sg[e, pl.ds(0, 16)] + sg[KS + e, pl.ds(0, 16)] + ae_v[e, :]
            a = jnp.where(a > 0, a, 0.2 * a)
            exv = jnp.exp(a)
            ex_v[e, :] = exv
            ex128[e, pl.ds(0, 16)] = exv
            return 0

        lax.fori_loop(0, KS, erow, 0)
        pltpu.sync_copy(ex_v, ex_h.at[pl.ds(base, KS)])
        pltpu.sync_copy(ex128, dacc.at[dst_v], add=True)
        return 0

    lax.fori_loop(0, NCHS, chunk, 0)
    plsc.subcore_barrier()
    pltpu.sync_copy(dacc.at[pl.ds(sid * NR, NR)],
                    denp_h.at[cid, pl.ds(sid * NR, NR)])


def _msgs_sc_body(src_h, dst_h, ex_h, xw_h, outp_h,
                  src_v, dst_v, ex_v, xwg, zbuf, acc, s3):
    cid = lax.axis_index("c")
    sid = lax.axis_index("s")
    wid = cid * NS + sid

    def zrow(i, _):
        for j in range(D // 16):
            zbuf[i, pl.ds(j * 16, 16)] = jnp.zeros((16,), F32)
        return 0

    lax.fori_loop(0, 128, zrow, 0)
    for k2 in range(NR // 128):
        pltpu.sync_copy(zbuf, acc.at[pl.ds(sid * NR + k2 * 128, 128)])
    plsc.subcore_barrier()

    def chunk(t, _):
        base = wid * EPW + t * K
        pltpu.sync_copy(src_h.at[pl.ds(base, K)], src_v)
        pltpu.sync_copy(dst_h.at[pl.ds(base, K)], dst_v)
        pltpu.sync_copy(ex_h.at[pl.ds(base, K)], ex_v)
        c3 = pltpu.async_copy(xw_h.at[src_v], xwg, s3)
        c3.wait()

        def mrow(e, _):
            ex16 = ex_v[e, :]
            for j in range(H):
                bj = _lane_splat(ex16, j)
                xwg[e, pl.ds(j * 16, 16)] = xwg[e, pl.ds(j * 16, 16)] * bj
            return 0

        lax.fori_loop(0, K, mrow, 0)
        pltpu.sync_copy(xwg, acc.at[dst_v], add=True)
        return 0

    lax.fori_loop(0, NCH, chunk, 0)
    plsc.subcore_barrier()
    pltpu.sync_copy(acc.at[pl.ds(sid * NR, NR)],
                    outp_h.at[cid, pl.ds(sid * NR, NR)])


def _sc_mesh():
    return plsc.VectorSubcoreMesh(core_axis_name="c", subcore_axis_name="s",
                                  num_cores=NC, num_subcores=NS)


@functools.cache
def _get_scores_sc():
    return functools.partial(
        pl.kernel,
        mesh=_sc_mesh(),
        out_type=[jax.ShapeDtypeStruct((N_EDGES, 16), F32),
                  jax.ShapeDtypeStruct((NC, NP, D), F32)],
        scratch_types=[
            pltpu.VMEM((2 * KS,), jnp.int32), pltpu.VMEM((KS,), jnp.int32),
            pltpu.VMEM((KS, 16), F32), pltpu.VMEM((2 * KS, D), F32),
            pltpu.VMEM((KS, 16), F32), pltpu.VMEM((KS, D), F32),
            pltpu.VMEM((128, D), F32),
            pltpu.VMEM_SHARED((NP, D), F32),
            pltpu.SemaphoreType.DMA,
        ],
    )(_scores_sc_body)


@functools.cache
def _get_msgs_sc():
    return functools.partial(
        pl.kernel,
        mesh=_sc_mesh(),
        out_type=jax.ShapeDtypeStruct((NC, NP, D), F32),
        scratch_types=[
            pltpu.VMEM((K,), jnp.int32), pltpu.VMEM((K,), jnp.int32),
            pltpu.VMEM((K, 16), F32), pltpu.VMEM((K, D), F32),
            pltpu.VMEM((128, D), F32),
            pltpu.VMEM_SHARED((NP, D), F32),
            pltpu.SemaphoreType.DMA,
        ],
    )(_msgs_sc_body)


def _att_mat(a):
    m = (a[:, :, None] * jnp.eye(H, dtype=a.dtype)[:, None, :]).reshape(D, H)
    return jnp.concatenate([m, m], axis=1)


def kernel(node_features, edge_index, edge_features, W_np, b_np, g_np,
           beta_np, W_ep, b_ep, g_ep, beta_ep, Wg, att_s, att_d, W_e, att_e,
           bias_g, lnG, lnB):
    src = edge_index[0]
    dst = edge_index[1]
    row = lambda v: v[None, :]

    pad = lambda m: jnp.concatenate([m, jnp.zeros((D, D - 16), F32)], axis=1)
    Ps = [pad(_att_mat(att_s[i])) for i in range(2)]
    Pd = [pad(_att_mat(att_d[i])) for i in range(2)]
    Ve = [jnp.dot(W_e[i], _att_mat(att_e[i])) for i in range(2)]
    Ex16 = jnp.concatenate(
        [jnp.repeat(jnp.eye(H, dtype=F32), D // H, axis=1),
         jnp.zeros((H, D), F32)], axis=0)

    xw0, as0, ad0 = _node_prep(node_features, W_np, row(b_np), row(g_np),
                               row(beta_np), Wg[0], Ps[0], Pd[0])
    ae0, ae1 = _edge_prep(edge_features, W_ep, row(b_ep), row(g_ep),
                          row(beta_ep), Ve[0], Ve[1])

    scores_sc = _get_scores_sc()
    msgs_sc = _get_msgs_sc()
    dstN = dst + N_NODES
    T0 = jnp.concatenate([as0, ad0], axis=0)
    ex0, denp0 = scores_sc(src, dstN, dst, ae0, T0)
    outp0 = msgs_sc(src, dst, ex0, xw0)
    xw1, as1, ad1 = _finish0(outp0, denp0, Ex16, row(bias_g[0]), row(lnG[0]),
                             row(lnB[0]), Wg[1], Ps[1], Pd[1])

    T1 = jnp.concatenate([as1, ad1], axis=0)
    ex1, denp1 = scores_sc(src, dstN, dst, ae1, T1)
    outp1 = msgs_sc(src, dst, ex1, xw1)
    x, gemb = _finish1(outp1, denp1, Ex16, row(bias_g[1]), row(lnG[1]),
                       row(lnB[1]))
    return (x, gemb)


# speedup vs baseline: 25.0719x; 1.1036x over previous
import functools

import jax
import jax.numpy as jnp
from jax import lax
from jax.experimental import pallas as pl
from jax.experimental.pallas import tpu as pltpu
from jax.experimental.pallas import tpu_sc as plsc

F32 = jnp.float32

N_NODES = 10000
N_EDGES = 320000
D = 128
H = 8
NC, NS = 2, 16
NW = NC * NS
EPW = N_EDGES // NW
K = 80
NCH = EPW // K
NP = 10240
NR = NP // NS


def _ln(h, g, b):
    m = jnp.mean(h, axis=-1, keepdims=True)
    v = jnp.mean((h - m) ** 2, axis=-1, keepdims=True)
    return (h - m) * lax.rsqrt(v + 1e-5) * g + b


def _node_prep_body(nf, W, b, g, be, Wg, Ps, Pd, xw_o, as_o, ad_o):
    x = jnp.maximum(_ln(jnp.dot(nf[...], W[...], preferred_element_type=F32)
                        + b[...], g[...], be[...]), 0.0)
    xw = jnp.dot(x, Wg[...], preferred_element_type=F32)
    xw_o[...] = xw
    as_o[...] = jnp.dot(xw, Ps[...], preferred_element_type=F32)
    ad_o[...] = jnp.dot(xw, Pd[...], preferred_element_type=F32)


def _node_prep(nf, W, b, g, be, Wg, Ps16, Pd16):
    R = 400
    grid = (N_NODES // R,)
    full = lambda s: pl.BlockSpec(s, lambda i: (0, 0))
    return pl.pallas_call(
        _node_prep_body,
        grid=grid,
        in_specs=[pl.BlockSpec((R, D), lambda i: (i, 0)),
                  full((D, D)), full((1, D)), full((1, D)), full((1, D)),
                  full((D, D)), full((D, D)), full((D, D))],
        out_specs=[pl.BlockSpec((R, D), lambda i: (i, 0)),
                   pl.BlockSpec((R, D), lambda i: (i, 0)),
                   pl.BlockSpec((R, D), lambda i: (i, 0))],
        out_shape=[jax.ShapeDtypeStruct((N_NODES, D), F32),
                   jax.ShapeDtypeStruct((N_NODES, D), F32),
                   jax.ShapeDtypeStruct((N_NODES, D), F32)],
    )(nf, W, b, g, be, Wg, Ps16, Pd16)


def _edge_prep_body(ef, W, b, g, be, V0, V1, a0_o, a1_o):
    ea = jnp.maximum(_ln(jnp.dot(ef[...], W[...], preferred_element_type=F32)
                         + b[...], g[...], be[...]), 0.0)
    a0_o[...] = jnp.dot(ea, V0[...], preferred_element_type=F32)
    a1_o[...] = jnp.dot(ea, V1[...], preferred_element_type=F32)


def _edge_prep(ef, W, b, g, be, V016, V116):
    R = 640
    grid = (N_EDGES // R,)
    DE = ef.shape[1]
    full = lambda s: pl.BlockSpec(s, lambda i: (0, 0))
    return pl.pallas_call(
        _edge_prep_body,
        grid=grid,
        in_specs=[pl.BlockSpec((R, DE), lambda i: (i, 0)),
                  full((DE, D)), full((1, D)), full((1, D)), full((1, D)),
                  full((D, 16)), full((D, 16))],
        out_specs=[pl.BlockSpec((R, 16), lambda i: (i, 0)),
                   pl.BlockSpec((R, 16), lambda i: (i, 0))],
        out_shape=[jax.ShapeDtypeStruct((N_EDGES, 16), F32),
                   jax.ShapeDtypeStruct((N_EDGES, 16), F32)],
    )(ef, W, b, g, be, V016, V116)


def _finish0_body(op, dp, Ex, bias, g, be, Wg, Ps, Pd, xw_o, as_o, ad_o):
    p = op[0] + op[1]
    den16 = (dp[0] + dp[1])[:, :16]
    den = jnp.dot(den16, Ex[...], preferred_element_type=F32) + 1e-16
    x1 = jnp.maximum(_ln(p / den + bias[...], g[...], be[...]), 0.0)
    xin = x1 + x1
    xw = jnp.dot(xin, Wg[...], preferred_element_type=F32)
    xw_o[...] = xw
    as_o[...] = jnp.dot(xw, Ps[...], preferred_element_type=F32)
    ad_o[...] = jnp.dot(xw, Pd[...], preferred_element_type=F32)


def _finish0(outp, denp, Ex16, bias, g, be, Wg, Ps16, Pd16):
    R = 400
    grid = (N_NODES // R,)
    full = lambda s: pl.BlockSpec(s, lambda i: (0, 0))
    return pl.pallas_call(
        _finish0_body,
        grid=grid,
        in_specs=[pl.BlockSpec((NC, R, D), lambda i: (0, i, 0)),
                  pl.BlockSpec((NC, R, D), lambda i: (0, i, 0)),
                  full((16, D)), full((1, D)), full((1, D)), full((1, D)),
                  full((D, D)), full((D, D)), full((D, D))],
        out_specs=[pl.BlockSpec((R, D), lambda i: (i, 0)),
                   pl.BlockSpec((R, D), lambda i: (i, 0)),
                   pl.BlockSpec((R, D), lambda i: (i, 0))],
        out_shape=[jax.ShapeDtypeStruct((N_NODES, D), F32),
                   jax.ShapeDtypeStruct((N_NODES, D), F32),
                   jax.ShapeDtypeStruct((N_NODES, D), F32)],
    )(outp, denp, Ex16, bias, g, be, Wg, Ps16, Pd16)


def _finish1_body(op, dp, Ex, bias, g, be, x_o, gemb_o):
    p = op[0] + op[1]
    den16 = (dp[0] + dp[1])[:, :16]
    den = jnp.dot(den16, Ex[...], preferred_element_type=F32) + 1e-16
    x = _ln(p / den + bias[...], g[...], be[...])
    x_o[...] = x

    @pl.when(pl.program_id(0) == 0)
    def _():
        gemb_o[...] = jnp.zeros_like(gemb_o)

    gemb_o[...] += jnp.sum(x, axis=0, keepdims=True) * (1.0 / N_NODES)


def _finish1(outp, denp, Ex16, bias, g, be):
    R = 400
    grid = (N_NODES // R,)
    full = lambda s: pl.BlockSpec(s, lambda i: (0, 0))
    return pl.pallas_call(
        _finish1_body,
        grid=grid,
        in_specs=[pl.BlockSpec((NC, R, D), lambda i: (0, i, 0)),
                  pl.BlockSpec((NC, R, D), lambda i: (0, i, 0)),
                  full((16, D)), full((1, D)), full((1, D)), full((1, D))],
        out_specs=[pl.BlockSpec((R, D), lambda i: (i, 0)),
                   pl.BlockSpec((1, D), lambda i: (0, 0))],
        out_shape=[jax.ShapeDtypeStruct((N_NODES, D), F32),
                   jax.ShapeDtypeStruct((1, D), F32)],
    )(outp, denp, Ex16, bias, g, be)


_GATHER_DNUMS = lax.GatherDimensionNumbers(
    offset_dims=(), collapsed_slice_dims=(0,), start_index_map=(0,))


def _lane_splat(v, j):
    idx = jnp.full((16, 1), j, jnp.int32)
    return lax.gather(v, idx, _GATHER_DNUMS, (1,),
                      mode=lax.GatherScatterMode.PROMISE_IN_BOUNDS)

KS = 80
NCHS = EPW // KS


def _scores_sc_body(src_h, dst_h, ae_h, Ts_h, Td_h, ex_h,
                    src_v, dst_v, ae_v, sg1, sg2, ex_v, s1, s2):
    cid = lax.axis_index("c")
    sid = lax.axis_index("s")
    wid = cid * NS + sid

    def chunk(t, _):
        base = wid * EPW + t * KS
        pltpu.sync_copy(src_h.at[pl.ds(base, KS)], src_v)
        pltpu.sync_copy(dst_h.at[pl.ds(base, KS)], dst_v)
        pltpu.sync_copy(ae_h.at[pl.ds(base, KS)], ae_v)
        c1 = pltpu.async_copy(Ts_h.at[src_v], sg1, s1)
        c2 = pltpu.async_copy(Td_h.at[dst_v], sg2, s2)
        c1.wait()
        c2.wait()

        def erow(e, _):
            a = sg1[e, pl.ds(0, 16)] + sg2[e, pl.ds(0, 16)] + ae_v[e, :]
            a = jnp.where(a > 0, a, 0.2 * a)
            ex_v[e, :] = jnp.exp(a)
            return 0

        lax.fori_loop(0, KS, erow, 0)
        pltpu.sync_copy(ex_v, ex_h.at[pl.ds(base, KS)])
        return 0

    lax.fori_loop(0, NCHS, chunk, 0)


def _msgs_sc_body(src_h, dst_h, ex_h, xw_h, outp_h, denp_h,
                  src_v, dst_v, ex_v, xwg, ex128, zbuf, acc, s3):
    cid = lax.axis_index("c")
    sid = lax.axis_index("s")
    wid = cid * NS + sid

    def zrow(i, _):
        for j in range(D // 16):
            zbuf[i, pl.ds(j * 16, 16)] = jnp.zeros((16,), F32)
        return 0

    lax.fori_loop(0, 16, zrow, 0)

    def zrow2(i, _):
        for j in range(1, D // 16):
            ex128[i, pl.ds(j * 16, 16)] = jnp.zeros((16,), F32)
        return 0

    lax.fori_loop(0, K, zrow2, 0)

    def zacc():
        for k2 in range(NR // 16):
            pltpu.sync_copy(zbuf, acc.at[pl.ds(sid * NR + k2 * 16, 16)])

    zacc()
    plsc.subcore_barrier()

    def chunk(t, _):
        base = wid * EPW + t * K
        pltpu.sync_copy(src_h.at[pl.ds(base, K)], src_v)
        pltpu.sync_copy(dst_h.at[pl.ds(base, K)], dst_v)
        pltpu.sync_copy(ex_h.at[pl.ds(base, K)], ex_v)
        c3 = pltpu.async_copy(xw_h.at[src_v], xwg, s3)
        c3.wait()

        def mrow(e, _):
            ex16 = ex_v[e, :]
            for j in range(H):
                bj = _lane_splat(ex16, j)
                xwg[e, pl.ds(j * 16, 16)] = xwg[e, pl.ds(j * 16, 16)] * bj
            return 0

        lax.fori_loop(0, K, mrow, 0)
        pltpu.sync_copy(xwg, acc.at[dst_v], add=True)
        return 0

    lax.fori_loop(0, NCH, chunk, 0)
    plsc.subcore_barrier()
    pltpu.sync_copy(acc.at[pl.ds(sid * NR, NR)],
                    outp_h.at[cid, pl.ds(sid * NR, NR)])
    zacc()
    plsc.subcore_barrier()

    def chunk2(t, _):
        base = wid * EPW + t * K
        pltpu.sync_copy(dst_h.at[pl.ds(base, K)], dst_v)
        pltpu.sync_copy(ex_h.at[pl.ds(base, K)], ex_v)

        def erow(e, _):
            ex128[e, pl.ds(0, 16)] = ex_v[e, :]
            return 0

        lax.fori_loop(0, K, erow, 0)
        pltpu.sync_copy(ex128, acc.at[dst_v], add=True)
        return 0

    lax.fori_loop(0, NCH, chunk2, 0)
    plsc.subcore_barrier()
    pltpu.sync_copy(acc.at[pl.ds(sid * NR, NR)],
                    denp_h.at[cid, pl.ds(sid * NR, NR)])


def _sc_mesh():
    return plsc.VectorSubcoreMesh(core_axis_name="c", subcore_axis_name="s",
                                  num_cores=NC, num_subcores=NS)


@functools.cache
def _get_scores_sc():
    return functools.partial(
        pl.kernel,
        mesh=_sc_mesh(),
        out_type=jax.ShapeDtypeStruct((N_EDGES, 16), F32),
        scratch_types=[
            pltpu.VMEM((KS,), jnp.int32), pltpu.VMEM((KS,), jnp.int32),
            pltpu.VMEM((KS, 16), F32), pltpu.VMEM((KS, D), F32),
            pltpu.VMEM((KS, D), F32), pltpu.VMEM((KS, 16), F32),
            pltpu.SemaphoreType.DMA, pltpu.SemaphoreType.DMA,
        ],
    )(_scores_sc_body)


@functools.cache
def _get_msgs_sc():
    return functools.partial(
        pl.kernel,
        mesh=_sc_mesh(),
        out_type=[jax.ShapeDtypeStruct((NC, NP, D), F32),
                  jax.ShapeDtypeStruct((NC, NP, D), F32)],
        scratch_types=[
            pltpu.VMEM((K,), jnp.int32), pltpu.VMEM((K,), jnp.int32),
            pltpu.VMEM((K, 16), F32), pltpu.VMEM((K, D), F32),
            pltpu.VMEM((K, D), F32), pltpu.VMEM((16, D), F32),
            pltpu.VMEM_SHARED((NP, D), F32),
            pltpu.SemaphoreType.DMA,
        ],
    )(_msgs_sc_body)


def _att_mat(a):
    m = (a[:, :, None] * jnp.eye(H, dtype=a.dtype)[:, None, :]).reshape(D, H)
    return jnp.concatenate([m, m], axis=1)


def kernel(node_features, edge_index, edge_features, W_np, b_np, g_np,
           beta_np, W_ep, b_ep, g_ep, beta_ep, Wg, att_s, att_d, W_e, att_e,
           bias_g, lnG, lnB):
    src = edge_index[0]
    dst = edge_index[1]
    row = lambda v: v[None, :]

    pad = lambda m: jnp.concatenate([m, jnp.zeros((D, D - 16), F32)], axis=1)
    Ps = [pad(_att_mat(att_s[i])) for i in range(2)]
    Pd = [pad(_att_mat(att_d[i])) for i in range(2)]
    Ve = [jnp.dot(W_e[i], _att_mat(att_e[i])) for i in range(2)]
    Ex16 = jnp.concatenate(
        [jnp.repeat(jnp.eye(H, dtype=F32), D // H, axis=1),
         jnp.zeros((H, D), F32)], axis=0)

    xw0, as0, ad0 = _node_prep(node_features, W_np, row(b_np), row(g_np),
                               row(beta_np), Wg[0], Ps[0], Pd[0])
    ae0, ae1 = _edge_prep(edge_features, W_ep, row(b_ep), row(g_ep),
                          row(beta_ep), Ve[0], Ve[1])

    scores_sc = _get_scores_sc()
    msgs_sc = _get_msgs_sc()
    ex0 = scores_sc(src, dst, ae0, as0, ad0)
    outp0, denp0 = msgs_sc(src, dst, ex0, xw0)
    xw1, as1, ad1 = _finish0(outp0, denp0, Ex16, row(bias_g[0]), row(lnG[0]),
                             row(lnB[0]), Wg[1], Ps[1], Pd[1])

    ex1 = scores_sc(src, dst, ae1, as1, ad1)
    outp1, denp1 = msgs_sc(src, dst, ex1, xw1)
    x, gemb = _finish1(outp1, denp1, Ex16, row(bias_g[1]), row(lnG[1]),
                       row(lnB[1]))
    return (x, gemb)
